# baseline (speedup 1.0000x reference)
import functools

import jax
import jax.numpy as jnp
from jax import lax
from jax.experimental import pallas as pl
from jax.experimental.pallas import tpu as pltpu
from jax.experimental.pallas import tpu_sc as plsc

EMBED_DIM = 16
BATCH = 16384
NUM_CORES = 2
NUM_SUBCORES = 16
NUM_WORKERS = NUM_CORES * NUM_SUBCORES
BPW = BATCH // NUM_WORKERS
EPS = 1e-5


def _sc_gather_body(ids_hbm, table_hbm, t1_hbm, t2_hbm, sd_hbm,
                    ids_v, idx1_v, idx2_v, sd_v, rows1_v, rows2_v,
                    sem1, sem2):
    wid = lax.axis_index("s") * NUM_CORES + lax.axis_index("c")
    base = wid * BPW
    pltpu.sync_copy(ids_hbm.at[pl.ds(base, BPW), :], ids_v)
    lanes = lax.iota(jnp.int32, 16)

    def extract(k, _):
        rows = k * 16 + lanes
        c0 = plsc.load_gather(ids_v, [rows, lanes * 0])
        c1 = plsc.load_gather(ids_v, [rows, lanes * 0 + 1])
        c2 = plsc.load_gather(ids_v, [rows, lanes * 0 + 2])
        idx1_v[pl.ds(k * 16, 16)] = c0.astype(jnp.int32)
        idx2_v[pl.ds(k * 16, 16)] = c1.astype(jnp.int32)
        sd_v[pl.ds(k * 16, 16)] = c2
        return 0

    lax.fori_loop(0, BPW // 16, extract, 0, unroll=4)

    cp1 = pltpu.async_copy(table_hbm.at[idx1_v], rows1_v, sem1)
    cp2 = pltpu.async_copy(table_hbm.at[idx2_v], rows2_v, sem2)
    pltpu.sync_copy(sd_v, sd_hbm.at[pl.ds(base, BPW)])
    cp1.wait()
    pltpu.sync_copy(rows1_v, t1_hbm.at[pl.ds(base, BPW)])
    cp2.wait()
    pltpu.sync_copy(rows2_v, t2_hbm.at[pl.ds(base, BPW)])


@jax.jit
def _sc_gather(ids, table):
    mesh = plsc.VectorSubcoreMesh(core_axis_name="c", subcore_axis_name="s")
    fn = functools.partial(
        pl.kernel,
        mesh=mesh,
        out_type=[
            jax.ShapeDtypeStruct((BATCH, EMBED_DIM), jnp.float32),
            jax.ShapeDtypeStruct((BATCH, EMBED_DIM), jnp.float32),
            jax.ShapeDtypeStruct((BATCH,), jnp.float32),
        ],
        scratch_types=[
            pltpu.VMEM((BPW, 3), jnp.float32),
            pltpu.VMEM((BPW,), jnp.int32),
            pltpu.VMEM((BPW,), jnp.int32),
            pltpu.VMEM((BPW,), jnp.float32),
            pltpu.VMEM((BPW, EMBED_DIM), jnp.float32),
            pltpu.VMEM((BPW, EMBED_DIM), jnp.float32),
            pltpu.SemaphoreType.DMA,
            pltpu.SemaphoreType.DMA,
        ],
        compiler_params=pltpu.CompilerParams(use_tc_tiling_on_sc=False,
                                             needs_layout_passes=False),
    )(_sc_gather_body)
    return fn(ids, table)


def _tc_classifier_body(t1_ref, t2_ref, sd_ref, g1_ref, g2_ref, b1_ref,
                        b2_ref, w1_ref, w2_ref, gsd_ref, bsd_ref, wsd_ref,
                        bias_ref, out_ref):
    t1 = t1_ref[...]
    t2 = t2_ref[...]
    sd = sd_ref[...]
    inv_b = 1.0 / BATCH

    m1 = jnp.sum(t1, axis=0, keepdims=True) * inv_b
    m2 = jnp.sum(t2, axis=0, keepdims=True) * inv_b
    msd = jnp.sum(sd, axis=0, keepdims=True) * inv_b
    c1 = t1 - m1
    c2 = t2 - m2
    csd = sd - msd
    v1 = jnp.sum(c1 * c1, axis=0, keepdims=True) * inv_b
    v2 = jnp.sum(c2 * c2, axis=0, keepdims=True) * inv_b
    vsd = jnp.sum(csd * csd, axis=0, keepdims=True) * inv_b

    s1 = g1_ref[...] * jax.lax.rsqrt(v1 + EPS)
    s2 = g2_ref[...] * jax.lax.rsqrt(v2 + EPS)
    ssd = gsd_ref[...] * jax.lax.rsqrt(vsd + EPS)

    f1 = c1 * s1 + b1_ref[...]
    f2 = c2 * s2 + b2_ref[...]
    fsd = csd * ssd + bsd_ref[...]

    l1 = jax.lax.dot(f1, w1_ref[...], preferred_element_type=jnp.float32)
    l2 = jax.lax.dot(f2, w2_ref[...], preferred_element_type=jnp.float32)
    logits = l1 + l2 + fsd * wsd_ref[...] + bias_ref[...]
    out_ref[...] = 1.0 / (1.0 + jnp.exp(-logits))


@jax.jit
def _tc_classifier(t1, t2, sd, g1, g2, b1, b2, w1, w2, gsd, bsd, wsd, bias):
    return pl.pallas_call(
        _tc_classifier_body,
        out_shape=jax.ShapeDtypeStruct((BATCH, 1), jnp.float32),
    )(t1, t2, sd, g1, g2, b1, b2, w1, w2, gsd, bsd, wsd, bias)


def kernel(idsTensor, table, gamma, beta, W, b):
    t1, t2, sd = _sc_gather(idsTensor, table)
    sd = sd.reshape(BATCH, 1)
    g1 = gamma[:EMBED_DIM].reshape(1, EMBED_DIM)
    g2 = gamma[EMBED_DIM:2 * EMBED_DIM].reshape(1, EMBED_DIM)
    b1 = beta[:EMBED_DIM].reshape(1, EMBED_DIM)
    b2 = beta[EMBED_DIM:2 * EMBED_DIM].reshape(1, EMBED_DIM)
    w1 = W[0, :EMBED_DIM].reshape(EMBED_DIM, 1)
    w2 = W[0, EMBED_DIM:2 * EMBED_DIM].reshape(EMBED_DIM, 1)
    gsd = gamma[2 * EMBED_DIM].reshape(1, 1)
    bsd = beta[2 * EMBED_DIM].reshape(1, 1)
    wsd = W[0, 2 * EMBED_DIM].reshape(1, 1)
    bias = b.reshape(1, 1)
    return _tc_classifier(t1, t2, sd, g1, g2, b1, b2, w1, w2, gsd, bsd, wsd,
                          bias)


# speedup vs baseline: 1.3798x; 1.3798x over previous
import functools

import jax
import jax.numpy as jnp
from jax import lax
from jax.experimental import pallas as pl
from jax.experimental.pallas import tpu as pltpu
from jax.experimental.pallas import tpu_sc as plsc

EMBED_DIM = 16
BATCH = 16384
NUM_CORES = 2
NUM_SUBCORES = 16
NUM_WORKERS = NUM_CORES * NUM_SUBCORES
BPW = BATCH // NUM_WORKERS
EPS = 1e-5


def _sc_gather_body(ids_hbm, table_hbm, t1_hbm, t2_hbm, sd_hbm,
                    ids_v, idx1_v, idx2_v, sd_v, rows1_v, rows2_v,
                    sem1, sem2):
    wid = lax.axis_index("s") * NUM_CORES + lax.axis_index("c")
    base = wid * BPW
    pltpu.sync_copy(ids_hbm.at[pl.ds(base, BPW), :], ids_v)
    lanes = lax.iota(jnp.int32, 16)

    def extract(k, _):
        rows = k * 16 + lanes
        c0 = plsc.load_gather(ids_v, [rows, lanes * 0])
        c1 = plsc.load_gather(ids_v, [rows, lanes * 0 + 1])
        c2 = plsc.load_gather(ids_v, [rows, lanes * 0 + 2])
        idx1_v[pl.ds(k * 16, 16)] = c0.astype(jnp.int32)
        idx2_v[pl.ds(k * 16, 16)] = c1.astype(jnp.int32)
        sd_v[pl.ds(k * 16, 16)] = c2
        return 0

    lax.fori_loop(0, BPW // 16, extract, 0, unroll=4)

    cp1 = pltpu.async_copy(table_hbm.at[idx1_v], rows1_v, sem1)
    cp2 = pltpu.async_copy(table_hbm.at[idx2_v], rows2_v, sem2)
    pltpu.sync_copy(sd_v, sd_hbm.at[pl.ds(base, BPW)])
    cp1.wait()
    pltpu.sync_copy(rows1_v, t1_hbm.at[pl.ds(base, BPW)])
    cp2.wait()
    pltpu.sync_copy(rows2_v, t2_hbm.at[pl.ds(base, BPW)])


@jax.jit
def _sc_gather(ids, table):
    mesh = plsc.VectorSubcoreMesh(core_axis_name="c", subcore_axis_name="s")
    fn = functools.partial(
        pl.kernel,
        mesh=mesh,
        out_type=[
            jax.ShapeDtypeStruct((BATCH, EMBED_DIM), jnp.float32),
            jax.ShapeDtypeStruct((BATCH, EMBED_DIM), jnp.float32),
            jax.ShapeDtypeStruct((BATCH,), jnp.float32),
        ],
        scratch_types=[
            pltpu.VMEM((BPW, 3), jnp.float32),
            pltpu.VMEM((BPW,), jnp.int32),
            pltpu.VMEM((BPW,), jnp.int32),
            pltpu.VMEM((BPW,), jnp.float32),
            pltpu.VMEM((BPW, EMBED_DIM), jnp.float32),
            pltpu.VMEM((BPW, EMBED_DIM), jnp.float32),
            pltpu.SemaphoreType.DMA,
            pltpu.SemaphoreType.DMA,
        ],
        compiler_params=pltpu.CompilerParams(use_tc_tiling_on_sc=False,
                                             needs_layout_passes=False),
    )(_sc_gather_body)
    return fn(ids, table)


ROWS = BATCH // 8


def _tc_classifier_body(t1_ref, t2_ref, sd_ref, g1_ref, g2_ref, b1_ref,
                        b2_ref, w1_ref, w2_ref, gsd_ref, bsd_ref, wsd_ref,
                        bias_ref, out_ref):
    t1 = t1_ref[...]
    t2 = t2_ref[...]
    sd = sd_ref[...]
    inv_b = 1.0 / BATCH

    ci = lax.broadcasted_iota(jnp.int32, (128, 128), 0) % EMBED_DIM
    cj = lax.broadcasted_iota(jnp.int32, (128, 128), 1) % EMBED_DIM
    p_mat = jnp.where(ci == cj, inv_b, 0.0)
    gi = lax.broadcasted_iota(jnp.int32, (128, 8), 0) // EMBED_DIM
    gj = lax.broadcasted_iota(jnp.int32, (128, 8), 1)
    g_mat = jnp.where(gi == gj, 1.0, 0.0)

    s1 = jnp.sum(t1, axis=0, keepdims=True)
    s2 = jnp.sum(t2, axis=0, keepdims=True)
    m1 = jax.lax.dot(s1, p_mat, preferred_element_type=jnp.float32)
    m2 = jax.lax.dot(s2, p_mat, preferred_element_type=jnp.float32)
    c1 = t1 - m1
    c2 = t2 - m2
    q1 = jnp.sum(c1 * c1, axis=0, keepdims=True)
    q2 = jnp.sum(c2 * c2, axis=0, keepdims=True)
    v1 = jax.lax.dot(q1, p_mat, preferred_element_type=jnp.float32)
    v2 = jax.lax.dot(q2, p_mat, preferred_element_type=jnp.float32)

    msd = jnp.sum(sd) * inv_b
    csd = sd - msd
    vsd = jnp.sum(csd * csd) * inv_b

    sw1 = g1_ref[...] * jax.lax.rsqrt(v1 + EPS) * w1_ref[...]
    sw2 = g2_ref[...] * jax.lax.rsqrt(v2 + EPS) * w2_ref[...]
    swsd = gsd_ref[0, 0] * jax.lax.rsqrt(vsd + EPS) * wsd_ref[0, 0]

    contrib1 = jax.lax.dot(c1 * sw1, g_mat,
                           preferred_element_type=jnp.float32)
    contrib2 = jax.lax.dot(c2 * sw2, g_mat,
                           preferred_element_type=jnp.float32)
    bw8 = jax.lax.dot(b1_ref[...] * w1_ref[...] + b2_ref[...] * w2_ref[...],
                      g_mat, preferred_element_type=jnp.float32)
    const8 = bw8 + bsd_ref[0, 0] * wsd_ref[0, 0] + bias_ref[0, 0]

    logits = contrib1 + contrib2 + csd * swsd + const8
    out_ref[...] = 1.0 / (1.0 + jnp.exp(-logits))


@jax.jit
def _tc_classifier(t1, t2, sd, g1, g2, b1, b2, w1, w2, gsd, bsd, wsd, bias):
    return pl.pallas_call(
        _tc_classifier_body,
        out_shape=jax.ShapeDtypeStruct((ROWS, 8), jnp.float32),
    )(t1, t2, sd, g1, g2, b1, b2, w1, w2, gsd, bsd, wsd, bias)


def kernel(idsTensor, table, gamma, beta, W, b):
    t1, t2, sd = _sc_gather(idsTensor, table)
    t1 = t1.reshape(ROWS, 128)
    t2 = t2.reshape(ROWS, 128)
    sd = sd.reshape(ROWS, 8)
    tile8 = lambda v: jnp.tile(v.reshape(1, EMBED_DIM), (1, 8))
    g1 = tile8(gamma[:EMBED_DIM])
    g2 = tile8(gamma[EMBED_DIM:2 * EMBED_DIM])
    b1 = tile8(beta[:EMBED_DIM])
    b2 = tile8(beta[EMBED_DIM:2 * EMBED_DIM])
    w1 = tile8(W[0, :EMBED_DIM])
    w2 = tile8(W[0, EMBED_DIM:2 * EMBED_DIM])
    gsd = gamma[2 * EMBED_DIM].reshape(1, 1)
    bsd = beta[2 * EMBED_DIM].reshape(1, 1)
    wsd = W[0, 2 * EMBED_DIM].reshape(1, 1)
    bias = b.reshape(1, 1)
    out = _tc_classifier(t1, t2, sd, g1, g2, b1, b2, w1, w2, gsd, bsd, wsd,
                         bias)
    return out.reshape(BATCH, 1)


# speedup vs baseline: 1.5711x; 1.1387x over previous
import functools

import jax
import jax.numpy as jnp
from jax import lax
from jax.experimental import pallas as pl
from jax.experimental.pallas import tpu as pltpu
from jax.experimental.pallas import tpu_sc as plsc

EMBED_DIM = 16
BATCH = 16384
NUM_CORES = 2
NUM_SUBCORES = 16
NUM_WORKERS = NUM_CORES * NUM_SUBCORES
BPW = BATCH // NUM_WORKERS
EPS = 1e-5


def _sc_gather_body(idx1_hbm, idx2_hbm, table_hbm, t1_hbm, t2_hbm,
                    idx1_v, idx2_v, rows1_v, rows2_v, sem1, sem2):
    wid = lax.axis_index("s") * NUM_CORES + lax.axis_index("c")
    base = wid * BPW
    pltpu.sync_copy(idx1_hbm.at[pl.ds(base, BPW)], idx1_v)
    pltpu.sync_copy(idx2_hbm.at[pl.ds(base, BPW)], idx2_v)
    cp1 = pltpu.async_copy(table_hbm.at[idx1_v], rows1_v, sem1)
    cp2 = pltpu.async_copy(table_hbm.at[idx2_v], rows2_v, sem2)
    cp1.wait()
    pltpu.sync_copy(rows1_v, t1_hbm.at[pl.ds(base, BPW)])
    cp2.wait()
    pltpu.sync_copy(rows2_v, t2_hbm.at[pl.ds(base, BPW)])


@jax.jit
def _sc_gather(idx1, idx2, table):
    mesh = plsc.VectorSubcoreMesh(core_axis_name="c", subcore_axis_name="s")
    fn = functools.partial(
        pl.kernel,
        mesh=mesh,
        out_type=[
            jax.ShapeDtypeStruct((BATCH, EMBED_DIM), jnp.float32),
            jax.ShapeDtypeStruct((BATCH, EMBED_DIM), jnp.float32),
        ],
        scratch_types=[
            pltpu.VMEM((BPW,), jnp.int32),
            pltpu.VMEM((BPW,), jnp.int32),
            pltpu.VMEM((BPW, EMBED_DIM), jnp.float32),
            pltpu.VMEM((BPW, EMBED_DIM), jnp.float32),
            pltpu.SemaphoreType.DMA,
            pltpu.SemaphoreType.DMA,
        ],
        compiler_params=pltpu.CompilerParams(use_tc_tiling_on_sc=False,
                                             needs_layout_passes=False),
    )(_sc_gather_body)
    return fn(idx1, idx2, table)


ROWS = BATCH // 8


def _tc_classifier_body(t1_ref, t2_ref, sd_ref, g1_ref, g2_ref, b1_ref,
                        b2_ref, w1_ref, w2_ref, gsd_ref, bsd_ref, wsd_ref,
                        bias_ref, out_ref):
    t1 = t1_ref[...]
    t2 = t2_ref[...]
    sd = sd_ref[...]
    inv_b = 1.0 / BATCH

    ci = lax.broadcasted_iota(jnp.int32, (128, 128), 0) % EMBED_DIM
    cj = lax.broadcasted_iota(jnp.int32, (128, 128), 1) % EMBED_DIM
    p_mat = jnp.where(ci == cj, inv_b, 0.0)
    gi = lax.broadcasted_iota(jnp.int32, (128, 8), 0) // EMBED_DIM
    gj = lax.broadcasted_iota(jnp.int32, (128, 8), 1)
    g_mat = jnp.where(gi == gj, 1.0, 0.0)

    s1 = jnp.sum(t1, axis=0, keepdims=True)
    s2 = jnp.sum(t2, axis=0, keepdims=True)
    m1 = jax.lax.dot(s1, p_mat, preferred_element_type=jnp.float32)
    m2 = jax.lax.dot(s2, p_mat, preferred_element_type=jnp.float32)
    c1 = t1 - m1
    c2 = t2 - m2
    q1 = jnp.sum(c1 * c1, axis=0, keepdims=True)
    q2 = jnp.sum(c2 * c2, axis=0, keepdims=True)
    v1 = jax.lax.dot(q1, p_mat, preferred_element_type=jnp.float32)
    v2 = jax.lax.dot(q2, p_mat, preferred_element_type=jnp.float32)

    msd = jnp.sum(sd) * inv_b
    csd = sd - msd
    vsd = jnp.sum(csd * csd) * inv_b

    sw1 = g1_ref[...] * jax.lax.rsqrt(v1 + EPS) * w1_ref[...]
    sw2 = g2_ref[...] * jax.lax.rsqrt(v2 + EPS) * w2_ref[...]
    swsd = gsd_ref[0, 0] * jax.lax.rsqrt(vsd + EPS) * wsd_ref[0, 0]

    contrib1 = jax.lax.dot(c1 * sw1, g_mat,
                           preferred_element_type=jnp.float32)
    contrib2 = jax.lax.dot(c2 * sw2, g_mat,
                           preferred_element_type=jnp.float32)
    bw8 = jax.lax.dot(b1_ref[...] * w1_ref[...] + b2_ref[...] * w2_ref[...],
                      g_mat, preferred_element_type=jnp.float32)
    const8 = bw8 + bsd_ref[0, 0] * wsd_ref[0, 0] + bias_ref[0, 0]

    logits = contrib1 + contrib2 + csd * swsd + const8
    out_ref[...] = 1.0 / (1.0 + jnp.exp(-logits))


@jax.jit
def _tc_classifier(t1, t2, sd, g1, g2, b1, b2, w1, w2, gsd, bsd, wsd, bias):
    return pl.pallas_call(
        _tc_classifier_body,
        out_shape=jax.ShapeDtypeStruct((ROWS, 8), jnp.float32),
    )(t1, t2, sd, g1, g2, b1, b2, w1, w2, gsd, bsd, wsd, bias)


def kernel(idsTensor, table, gamma, beta, W, b):
    idx1 = idsTensor[:, 0].astype(jnp.int32)
    idx2 = idsTensor[:, 1].astype(jnp.int32)
    sd = idsTensor[:, 2].reshape(ROWS, 8)
    t1, t2 = _sc_gather(idx1, idx2, table)
    t1 = t1.reshape(ROWS, 128)
    t2 = t2.reshape(ROWS, 128)
    tile8 = lambda v: jnp.tile(v.reshape(1, EMBED_DIM), (1, 8))
    g1 = tile8(gamma[:EMBED_DIM])
    g2 = tile8(gamma[EMBED_DIM:2 * EMBED_DIM])
    b1 = tile8(beta[:EMBED_DIM])
    b2 = tile8(beta[EMBED_DIM:2 * EMBED_DIM])
    w1 = tile8(W[0, :EMBED_DIM])
    w2 = tile8(W[0, EMBED_DIM:2 * EMBED_DIM])
    gsd = gamma[2 * EMBED_DIM].reshape(1, 1)
    bsd = beta[2 * EMBED_DIM].reshape(1, 1)
    wsd = W[0, 2 * EMBED_DIM].reshape(1, 1)
    bias = b.reshape(1, 1)
    out = _tc_classifier(t1, t2, sd, g1, g2, b1, b2, w1, w2, gsd, bsd, wsd,
                         bias)
    return out.reshape(BATCH, 1)


# speedup vs baseline: 2.3155x; 1.4738x over previous
import functools

import jax
import jax.numpy as jnp
from jax import lax
from jax.experimental import pallas as pl
from jax.experimental.pallas import tpu as pltpu
from jax.experimental.pallas import tpu_sc as plsc

EMBED_DIM = 16
BATCH = 16384
NTEAMS = 100000
NUM_CORES = 2
NUM_SUBCORES = 16
NUM_WORKERS = NUM_CORES * NUM_SUBCORES
BPW = BATCH // NUM_WORKERS
EPS = 1e-5


def _sc_gather_body(idx1_hbm, idx2_hbm, ttf_hbm, t1_hbm, t2_hbm,
                    idx1_v, idx2_v, idxs1_v, idxs2_v, rows1_v, rows2_v,
                    sem1, sem2):
    wid = lax.axis_index("s") * NUM_CORES + lax.axis_index("c")
    base = wid * BPW
    pltpu.sync_copy(idx1_hbm.at[pl.ds(base, BPW)], idx1_v)
    pltpu.sync_copy(idx2_hbm.at[pl.ds(base, BPW)], idx2_v)

    def build(k, _):
        v1 = idx1_v[pl.ds(k * 16, 16)]
        v2 = idx2_v[pl.ds(k * 16, 16)]
        for j in range(EMBED_DIM):
            idxs1_v[j, pl.ds(k * 16, 16)] = v1 + (j * NTEAMS)
            idxs2_v[j, pl.ds(k * 16, 16)] = v2 + (j * NTEAMS)
        return 0

    lax.fori_loop(0, BPW // 16, build, 0, unroll=2)

    copies = []
    for j in range(EMBED_DIM):
        copies.append(
            pltpu.async_copy(ttf_hbm.at[idxs1_v.at[j]], rows1_v.at[j], sem1))
        copies.append(
            pltpu.async_copy(ttf_hbm.at[idxs2_v.at[j]], rows2_v.at[j], sem2))
    for cp in copies:
        cp.wait()
    pltpu.sync_copy(rows1_v, t1_hbm.at[:, pl.ds(base, BPW)])
    pltpu.sync_copy(rows2_v, t2_hbm.at[:, pl.ds(base, BPW)])


@jax.jit
def _sc_gather(idx1, idx2, ttf):
    mesh = plsc.VectorSubcoreMesh(core_axis_name="c", subcore_axis_name="s")
    fn = functools.partial(
        pl.kernel,
        mesh=mesh,
        out_type=[
            jax.ShapeDtypeStruct((EMBED_DIM, BATCH), jnp.float32),
            jax.ShapeDtypeStruct((EMBED_DIM, BATCH), jnp.float32),
        ],
        scratch_types=[
            pltpu.VMEM((BPW,), jnp.int32),
            pltpu.VMEM((BPW,), jnp.int32),
            pltpu.VMEM((EMBED_DIM, BPW), jnp.int32),
            pltpu.VMEM((EMBED_DIM, BPW), jnp.int32),
            pltpu.VMEM((EMBED_DIM, BPW), jnp.float32),
            pltpu.VMEM((EMBED_DIM, BPW), jnp.float32),
            pltpu.SemaphoreType.DMA,
            pltpu.SemaphoreType.DMA,
        ],
        compiler_params=pltpu.CompilerParams(use_tc_tiling_on_sc=False,
                                             needs_layout_passes=False),
    )(_sc_gather_body)
    return fn(idx1, idx2, ttf)


def _tc_classifier_body(t1_ref, t2_ref, sd_ref, par_ref, scal_ref, out_ref):
    inv_b = 1.0 / BATCH
    t1 = t1_ref[...].reshape(EMBED_DIM, 128, 128)
    t2 = t2_ref[...].reshape(EMBED_DIM, 128, 128)
    sd = sd_ref[...]

    m1 = jnp.sum(t1, axis=(1, 2), keepdims=True) * inv_b
    m2 = jnp.sum(t2, axis=(1, 2), keepdims=True) * inv_b
    c1 = t1 - m1
    c2 = t2 - m2
    v1 = jnp.sum(c1 * c1, axis=(1, 2), keepdims=True) * inv_b
    v2 = jnp.sum(c2 * c2, axis=(1, 2), keepdims=True) * inv_b

    par = par_ref[...]
    g1 = par[:, 0:1].reshape(EMBED_DIM, 1, 1)
    g2 = par[:, 1:2].reshape(EMBED_DIM, 1, 1)
    b1 = par[:, 2:3]
    b2 = par[:, 3:4]
    w1 = par[:, 4:5].reshape(EMBED_DIM, 1, 1)
    w2 = par[:, 5:6].reshape(EMBED_DIM, 1, 1)
    gsd = scal_ref[0, 0]
    bsd = scal_ref[0, 1]
    wsd = scal_ref[0, 2]
    bias = scal_ref[0, 3]

    sw1 = g1 * jax.lax.rsqrt(v1 + EPS) * w1
    sw2 = g2 * jax.lax.rsqrt(v2 + EPS) * w2

    msd = jnp.sum(sd) * inv_b
    csd = sd - msd
    vsd = jnp.sum(csd * csd) * inv_b
    swsd = gsd * jax.lax.rsqrt(vsd + EPS) * wsd

    const = (jnp.sum(b1 * par[:, 4:5]) + jnp.sum(b2 * par[:, 5:6])
             + bsd * wsd + bias)
    logits = (jnp.sum(c1 * sw1, axis=0) + jnp.sum(c2 * sw2, axis=0)
              + csd * swsd + const)
    out_ref[...] = 1.0 / (1.0 + jnp.exp(-logits))


@jax.jit
def _tc_classifier(t1p, t2p, sd, par, scal):
    return pl.pallas_call(
        _tc_classifier_body,
        out_shape=jax.ShapeDtypeStruct((128, 128), jnp.float32),
    )(t1p, t2p, sd, par, scal)


def kernel(idsTensor, table, gamma, beta, W, b):
    idx1 = idsTensor[:, 0].astype(jnp.int32)
    idx2 = idsTensor[:, 1].astype(jnp.int32)
    sd = idsTensor[:, 2].reshape(128, 128)
    ttf = table.T.reshape(EMBED_DIM * NTEAMS)
    t1, t2 = _sc_gather(idx1, idx2, ttf)
    t1p = t1.reshape(2048, 128)
    t2p = t2.reshape(2048, 128)
    par = jnp.stack(
        [gamma[:EMBED_DIM], gamma[EMBED_DIM:2 * EMBED_DIM],
         beta[:EMBED_DIM], beta[EMBED_DIM:2 * EMBED_DIM],
         W[0, :EMBED_DIM], W[0, EMBED_DIM:2 * EMBED_DIM]], axis=1)
    scal = jnp.stack(
        [gamma[2 * EMBED_DIM], beta[2 * EMBED_DIM], W[0, 2 * EMBED_DIM],
         b[0]]).reshape(1, 4)
    out = _tc_classifier(t1p, t2p, sd, par, scal)
    return out.reshape(BATCH, 1)
